# baseline (speedup 1.0000x reference)
import functools

import jax
import jax.numpy as jnp
from jax.experimental import pallas as pl
from jax.experimental.pallas import tpu as pltpu

CODE_DIM = 256
CODEBOOK_NUM = 8
CODEBOOK_SIZE = 512
SCODEBOOK_ROWS = 1024
B = 8
T = 1024

_DIST_PREC = jax.lax.Precision.DEFAULT
_EXACT_PREC = jax.lax.Precision.HIGHEST


def _rvq_kernel(z_ref, scb_ref, sizei_ref, zq_ref, mapped_ref,
                embt_scr, embsq_scr, embhi_scr, embmid_scr, emblo_scr):
    b = pl.program_id(0)

    @pl.when(b == 0)
    def _init():
        size_col = sizei_ref[...]
        riota = jax.lax.broadcasted_iota(jnp.int32,
                                         (CODEBOOK_SIZE, SCODEBOOK_ROWS), 1)
        osel = (riota == size_col).astype(jnp.float32)
        emb = jax.lax.dot_general(
            osel, scb_ref[...], (((1,), (0,)), ((), ())),
            precision=_EXACT_PREC, preferred_element_type=jnp.float32)
        embt_scr[...] = jnp.transpose(emb, (1, 0))
        embsq_scr[...] = jnp.transpose(
            jnp.sum(emb * emb, axis=1, keepdims=True), (1, 0))
        hi = emb.astype(jnp.bfloat16)
        r1 = emb - hi.astype(jnp.float32)
        mid = r1.astype(jnp.bfloat16)
        lo = (r1 - mid.astype(jnp.float32)).astype(jnp.bfloat16)
        embhi_scr[...] = hi
        embmid_scr[...] = mid
        emblo_scr[...] = lo

    x = jnp.transpose(z_ref[0], (1, 0))
    embt = embt_scr[...]
    emb_sq = embsq_scr[...]
    emb_hi = embhi_scr[...]
    emb_mid = embmid_scr[...]
    emb_lo = emblo_scr[...]
    size_row = sizei_ref[...].reshape(1, CODEBOOK_SIZE)

    residual = x
    zq = jnp.zeros_like(x)
    mapped_cols = []
    for q in range(CODEBOOK_NUM):
        m = jax.lax.dot_general(
            residual, embt, (((1,), (0,)), ((), ())),
            precision=_DIST_PREC, preferred_element_type=jnp.float32)
        rsq = jnp.sum(residual * residual, axis=1, keepdims=True)
        d = (rsq - 2.0 * m) + emb_sq
        idx = jnp.argmin(d, axis=1)
        kiota = jax.lax.broadcasted_iota(jnp.int32, (T, CODEBOOK_SIZE), 1)
        sel = kiota == idx[:, None]
        onehot = sel.astype(jnp.bfloat16)
        dn = (((1,), (0,)), ((), ()))
        quant = (jax.lax.dot_general(onehot, emb_hi, dn,
                                     preferred_element_type=jnp.float32)
                 + jax.lax.dot_general(onehot, emb_mid, dn,
                                       preferred_element_type=jnp.float32)
                 + jax.lax.dot_general(onehot, emb_lo, dn,
                                       preferred_element_type=jnp.float32))
        zq = zq + quant
        residual = residual - quant
        mapped_cols.append(jnp.sum(
            jnp.where(sel, size_row, 0), axis=1, keepdims=True))

    mapped_ref[0] = jnp.concatenate(mapped_cols, axis=1)

    zq_ref[0] = x + (zq - x)


@functools.partial(jax.jit, static_argnames=())
def kernel(z, scodebook, size):
    sizei = size.reshape(CODEBOOK_SIZE, 1)
    zq, mapped_btq = pl.pallas_call(
        _rvq_kernel,
        grid=(B,),
        in_specs=[
            pl.BlockSpec((1, CODE_DIM, T), lambda b: (b, 0, 0)),
            pl.BlockSpec((SCODEBOOK_ROWS, CODE_DIM), lambda b: (0, 0)),
            pl.BlockSpec((CODEBOOK_SIZE, 1), lambda b: (0, 0)),
        ],
        out_specs=[
            pl.BlockSpec((1, T, CODE_DIM), lambda b: (b, 0, 0)),
            pl.BlockSpec((1, T, CODEBOOK_NUM), lambda b: (b, 0, 0)),
        ],
        out_shape=[
            jax.ShapeDtypeStruct((B, T, CODE_DIM), jnp.float32),
            jax.ShapeDtypeStruct((B, T, CODEBOOK_NUM), jnp.int32),
        ],
        scratch_shapes=[
            pltpu.VMEM((CODE_DIM, CODEBOOK_SIZE), jnp.float32),
            pltpu.VMEM((1, CODEBOOK_SIZE), jnp.float32),
            pltpu.VMEM((CODEBOOK_SIZE, CODE_DIM), jnp.bfloat16),
            pltpu.VMEM((CODEBOOK_SIZE, CODE_DIM), jnp.bfloat16),
            pltpu.VMEM((CODEBOOK_SIZE, CODE_DIM), jnp.bfloat16),
        ],
    )(z, scodebook, sizei)
    return zq, jnp.transpose(mapped_btq, (2, 0, 1))


# speedup vs baseline: 1.2845x; 1.2845x over previous
import functools

import jax
import jax.numpy as jnp
from jax.experimental import pallas as pl
from jax.experimental.pallas import tpu as pltpu

CODE_DIM = 256
CODEBOOK_NUM = 8
CODEBOOK_SIZE = 512
SCODEBOOK_ROWS = 1024
B = 8
T = 1024

_DIST_PREC = jax.lax.Precision.DEFAULT
_EXACT_PREC = jax.lax.Precision.HIGHEST


def _rvq_kernel(z_ref, scb_ref, sizei_ref, zq_ref, mapped_ref,
                emb_scr, embsq_scr, embhi_scr, embmid_scr, emblo_scr):
    b = pl.program_id(0)

    @pl.when(b == 0)
    def _init():
        size_col = sizei_ref[...]
        riota = jax.lax.broadcasted_iota(jnp.int32,
                                         (CODEBOOK_SIZE, SCODEBOOK_ROWS), 1)
        osel = (riota == size_col).astype(jnp.float32)
        emb = jax.lax.dot_general(
            osel, scb_ref[...], (((1,), (0,)), ((), ())),
            precision=_EXACT_PREC, preferred_element_type=jnp.float32)
        emb_scr[...] = emb
        embsq_scr[...] = jnp.sum(emb * emb, axis=1, keepdims=True)
        hi = emb.astype(jnp.bfloat16)
        r1 = emb - hi.astype(jnp.float32)
        mid = r1.astype(jnp.bfloat16)
        lo = (r1 - mid.astype(jnp.float32)).astype(jnp.bfloat16)
        embhi_scr[...] = hi
        embmid_scr[...] = mid
        emblo_scr[...] = lo

    emb = emb_scr[...]
    emb_sq = embsq_scr[...]
    emb_hi = embhi_scr[...]
    emb_mid = embmid_scr[...]
    emb_lo = emblo_scr[...]
    size_col = sizei_ref[...]
    kiota = jax.lax.broadcasted_iota(jnp.int32, (CODEBOOK_SIZE, T), 0)
    dn = (((0,), (0,)), ((), ()))

    xs = [z_ref[0], z_ref[1]]
    residuals = list(xs)
    zqs = [jnp.zeros_like(xs[0]), jnp.zeros_like(xs[1])]
    mapped_rows = [[], []]
    for q in range(CODEBOOK_NUM):
        for j in range(2):
            m = jax.lax.dot_general(
                emb, residuals[j], (((1,), (0,)), ((), ())),
                precision=_DIST_PREC, preferred_element_type=jnp.float32)
            rsq = jnp.sum(residuals[j] * residuals[j], axis=0,
                          keepdims=True)
            d = (rsq - 2.0 * m) + emb_sq
            idx = jnp.argmin(d, axis=0)
            sel = kiota == idx[None, :]
            onehot = sel.astype(jnp.bfloat16)
            quant = (jax.lax.dot_general(emb_hi, onehot, dn,
                                         preferred_element_type=jnp.float32)
                     + jax.lax.dot_general(emb_mid, onehot, dn,
                                           preferred_element_type=jnp.float32)
                     + jax.lax.dot_general(emb_lo, onehot, dn,
                                           preferred_element_type=jnp.float32))
            zqs[j] = zqs[j] + quant
            residuals[j] = residuals[j] - quant
            mapped_rows[j].append(jnp.sum(
                jnp.where(sel, size_col, 0), axis=0, keepdims=True))

    for j in range(2):
        mapped_ref[j] = jnp.concatenate(mapped_rows[j], axis=0)
        zq_ref[j] = xs[j] + (zqs[j] - xs[j])


@functools.partial(jax.jit, static_argnames=())
def kernel(z, scodebook, size):
    sizei = size.reshape(CODEBOOK_SIZE, 1)
    zq_bdt, mapped = pl.pallas_call(
        _rvq_kernel,
        grid=(B // 2,),
        in_specs=[
            pl.BlockSpec((2, CODE_DIM, T), lambda b: (b, 0, 0)),
            pl.BlockSpec((SCODEBOOK_ROWS, CODE_DIM), lambda b: (0, 0)),
            pl.BlockSpec((CODEBOOK_SIZE, 1), lambda b: (0, 0)),
        ],
        out_specs=[
            pl.BlockSpec((2, CODE_DIM, T), lambda b: (b, 0, 0)),
            pl.BlockSpec((2, CODEBOOK_NUM, T), lambda b: (b, 0, 0)),
        ],
        out_shape=[
            jax.ShapeDtypeStruct((B, CODE_DIM, T), jnp.float32),
            jax.ShapeDtypeStruct((B, CODEBOOK_NUM, T), jnp.int32),
        ],
        scratch_shapes=[
            pltpu.VMEM((CODEBOOK_SIZE, CODE_DIM), jnp.float32),
            pltpu.VMEM((CODEBOOK_SIZE, 1), jnp.float32),
            pltpu.VMEM((CODEBOOK_SIZE, CODE_DIM), jnp.bfloat16),
            pltpu.VMEM((CODEBOOK_SIZE, CODE_DIM), jnp.bfloat16),
            pltpu.VMEM((CODEBOOK_SIZE, CODE_DIM), jnp.bfloat16),
        ],
    )(z, scodebook, sizei)
    zq = jnp.transpose(zq_bdt, (0, 2, 1))
    return zq, jnp.transpose(mapped, (1, 0, 2))
